# Initial kernel scaffold; baseline (speedup 1.0000x reference)
import jax
import jax.numpy as jnp
from jax.experimental import pallas as pl


def kernel(x, edge_index, edge_weight, W1, b1, W2, b2, W3, b3, W4, b4, W5, b5, W6, b6, W7, b7):
    raise NotImplementedError("write your pallas kernel here")


# speedup vs baseline: 7.4772x; 7.4772x over previous
import functools

import jax
import jax.numpy as jnp
from jax import lax
from jax.experimental import pallas as pl
from jax.experimental.pallas import tpu as pltpu
from jax.experimental.pallas import tpu_sc as plsc

N = 10000
NPAD = 10240
E = 320000
NC, NS, LANES = 2, 16, 16
NW = NC * NS
BLK = 128
NB = -(-E // (NW * BLK))
EPW = NB * BLK
E_PAD = EPW * NW
RPT = NPAD // NS
RB = 1280


def _sc_degree(dst_i, w_f):
    mesh = plsc.VectorSubcoreMesh(core_axis_name="c", subcore_axis_name="s")

    @functools.partial(
        pl.kernel,
        out_type=jax.ShapeDtypeStruct((NC, NPAD), jnp.float32),
        mesh=mesh,
        scratch_types=[
            pltpu.VMEM((BLK,), jnp.int32),
            pltpu.VMEM((BLK,), jnp.float32),
            pltpu.VMEM((RPT,), jnp.float32),
            pltpu.VMEM_SHARED((NPAD,), jnp.float32),
        ],
    )
    def deg_kernel(dst_ref, w_ref, out_ref, idx_v, w_v, z_v, acc_sh):
        cid = lax.axis_index("c")
        sid = lax.axis_index("s")
        wid = cid * NS + sid

        def zbody(i, _):
            z_v[pl.ds(i * LANES, LANES)] = jnp.zeros((LANES,), jnp.float32)
            return 0

        lax.fori_loop(0, RPT // LANES, zbody, 0)
        pltpu.sync_copy(z_v, acc_sh.at[pl.ds(sid * RPT, RPT)])
        plsc.subcore_barrier()

        base = wid * EPW

        def blk(j, _):
            off = base + j * BLK
            pltpu.sync_copy(dst_ref.at[pl.ds(off, BLK)], idx_v)
            pltpu.sync_copy(w_ref.at[pl.ds(off, BLK)], w_v)
            pltpu.sync_copy(w_v, acc_sh.at[idx_v], add=True)
            return 0

        lax.fori_loop(0, NB, blk, 0)
        plsc.subcore_barrier()
        pltpu.sync_copy(
            acc_sh.at[pl.ds(sid * RPT, RPT)],
            out_ref.at[cid, pl.ds(sid * RPT, RPT)],
        )

    return deg_kernel(dst_i, w_f)


def _sc_aggregate(g, src_i, dst_i, w_f, d):
    mesh = plsc.VectorSubcoreMesh(core_axis_name="c", subcore_axis_name="s")

    @functools.partial(
        pl.kernel,
        out_type=jax.ShapeDtypeStruct((NC, NPAD, d), jnp.float32),
        mesh=mesh,
        scratch_types=[
            pltpu.VMEM((BLK,), jnp.int32),
            pltpu.VMEM((BLK,), jnp.int32),
            pltpu.VMEM((BLK,), jnp.float32),
            pltpu.VMEM((BLK, d), jnp.float32),
            pltpu.VMEM((BLK, d), jnp.float32),
            pltpu.VMEM_SHARED((NPAD, d), jnp.float32),
            pltpu.SemaphoreType.DMA,
        ],
        compiler_params=pltpu.CompilerParams(use_tc_tiling_on_sc=False),
    )
    def agg_kernel(g_ref, src_ref, dst_ref, w_ref, out_ref,
                   si_v, di_v, w_v, rows_v, z_v, acc_sh, sem):
        cid = lax.axis_index("c")
        sid = lax.axis_index("s")
        wid = cid * NS + sid

        def zbody(i, _):
            for c in range(d // LANES):
                z_v[i, pl.ds(c * LANES, LANES)] = jnp.zeros((LANES,), jnp.float32)
            return 0

        lax.fori_loop(0, BLK, zbody, 0)
        for t in range(RPT // BLK):
            pltpu.sync_copy(z_v, acc_sh.at[pl.ds(sid * RPT + t * BLK, BLK)])
        plsc.subcore_barrier()

        base = wid * EPW

        def blk(j, _):
            off = base + j * BLK
            pltpu.sync_copy(src_ref.at[pl.ds(off, BLK)], si_v)
            pltpu.sync_copy(dst_ref.at[pl.ds(off, BLK)], di_v)
            pltpu.sync_copy(w_ref.at[pl.ds(off, BLK)], w_v)
            pltpu.async_copy(g_ref.at[si_v], rows_v, sem).wait()

            def ebody(gidx, _):
                wv = w_v[pl.ds(gidx * LANES, LANES)]
                for l in range(LANES):
                    i = gidx * LANES + l
                    s = wv[l]
                    for c in range(d // LANES):
                        sl = pl.ds(c * LANES, LANES)
                        rows_v[i, sl] = rows_v[i, sl] * s
                return 0

            lax.fori_loop(0, BLK // LANES, ebody, 0)
            pltpu.sync_copy(rows_v, acc_sh.at[di_v], add=True)
            return 0

        lax.fori_loop(0, NB, blk, 0)
        plsc.subcore_barrier()
        pltpu.sync_copy(
            acc_sh.at[pl.ds(sid * RPT, RPT)],
            out_ref.at[cid, pl.ds(sid * RPT, RPT)],
        )

    return agg_kernel(g, src_i, dst_i, w_f)


def _tc_prolog(degp, xp):
    def body(degp_ref, x_ref, dinv_ref, g_ref):
        deg = degp_ref[0] + degp_ref[1] + 1.0
        dinv = jnp.where(deg > 0, lax.rsqrt(deg), 0.0)
        dinv_ref[...] = dinv
        g_ref[...] = x_ref[...] * dinv

    return pl.pallas_call(
        body,
        grid=(NPAD // RB,),
        in_specs=[
            pl.BlockSpec((2, RB, 1), lambda i: (0, i, 0)),
            pl.BlockSpec((RB, 128), lambda i: (i, 0)),
        ],
        out_specs=[
            pl.BlockSpec((RB, 1), lambda i: (i, 0)),
            pl.BlockSpec((RB, 128), lambda i: (i, 0)),
        ],
        out_shape=[
            jax.ShapeDtypeStruct((NPAD, 1), jnp.float32),
            jax.ShapeDtypeStruct((NPAD, 128), jnp.float32),
        ],
    )(degp, xp)


def _tc_layer1(aggp, g, dinv, W1, b1, W2):
    def body(a_ref, g_ref, dinv_ref, w1_ref, b1_ref, w2_ref, out_ref):
        dinv = dinv_ref[...]
        s = dinv * (a_ref[0] + a_ref[1] + g_ref[...])
        h = jnp.maximum(
            jnp.dot(s, w1_ref[...], preferred_element_type=jnp.float32)
            + b1_ref[...], 0.0)
        out_ref[...] = dinv * jnp.dot(
            h, w2_ref[...], preferred_element_type=jnp.float32)

    return pl.pallas_call(
        body,
        grid=(NPAD // RB,),
        in_specs=[
            pl.BlockSpec((2, RB, 128), lambda i: (0, i, 0)),
            pl.BlockSpec((RB, 128), lambda i: (i, 0)),
            pl.BlockSpec((RB, 1), lambda i: (i, 0)),
            pl.BlockSpec((128, 256), lambda i: (0, 0)),
            pl.BlockSpec((1, 256), lambda i: (0, 0)),
            pl.BlockSpec((256, 128), lambda i: (0, 0)),
        ],
        out_specs=pl.BlockSpec((RB, 128), lambda i: (i, 0)),
        out_shape=jax.ShapeDtypeStruct((NPAD, 128), jnp.float32),
    )(aggp, g, dinv, W1, b1, W2)


def _tc_layer_mid(aggp, g, dinv, b, Wn):
    d = g.shape[1]
    dn = Wn.shape[1]

    def body(a_ref, g_ref, dinv_ref, b_ref, wn_ref, out_ref):
        dinv = dinv_ref[...]
        h = jnp.maximum(
            dinv * (a_ref[0] + a_ref[1] + g_ref[...]) + b_ref[...], 0.0)
        out_ref[...] = dinv * jnp.dot(
            h, wn_ref[...], preferred_element_type=jnp.float32)

    return pl.pallas_call(
        body,
        grid=(NPAD // RB,),
        in_specs=[
            pl.BlockSpec((2, RB, d), lambda i: (0, i, 0)),
            pl.BlockSpec((RB, d), lambda i: (i, 0)),
            pl.BlockSpec((RB, 1), lambda i: (i, 0)),
            pl.BlockSpec((1, d), lambda i: (0, 0)),
            pl.BlockSpec((d, dn), lambda i: (0, 0)),
        ],
        out_specs=pl.BlockSpec((RB, dn), lambda i: (i, 0)),
        out_shape=jax.ShapeDtypeStruct((NPAD, dn), jnp.float32),
    )(aggp, g, dinv, b, Wn)


def _tc_layer_last(aggp, g, dinv, b):
    d = g.shape[1]

    def body(a_ref, g_ref, dinv_ref, b_ref, out_ref):
        out_ref[...] = jnp.maximum(
            dinv_ref[...] * (a_ref[0] + a_ref[1] + g_ref[...]) + b_ref[...],
            0.0)

    return pl.pallas_call(
        body,
        grid=(NPAD // RB,),
        in_specs=[
            pl.BlockSpec((2, RB, d), lambda i: (0, i, 0)),
            pl.BlockSpec((RB, d), lambda i: (i, 0)),
            pl.BlockSpec((RB, 1), lambda i: (i, 0)),
            pl.BlockSpec((1, d), lambda i: (0, 0)),
        ],
        out_specs=pl.BlockSpec((RB, d), lambda i: (i, 0)),
        out_shape=jax.ShapeDtypeStruct((NPAD, d), jnp.float32),
    )(aggp, g, dinv, b)


def kernel(x, edge_index, edge_weight, W1, b1, W2, b2, W3, b3, W4, b4,
           W5, b5, W6, b6, W7, b7):
    src = edge_index[0].astype(jnp.int32)
    dst = edge_index[1].astype(jnp.int32)
    w = edge_weight.astype(jnp.float32)
    pad = E_PAD - E
    src = jnp.concatenate([src, jnp.zeros((pad,), jnp.int32)])
    dst = jnp.concatenate([dst, jnp.zeros((pad,), jnp.int32)])
    w = jnp.concatenate([w, jnp.zeros((pad,), jnp.float32)])
    xp = jnp.pad(x, ((0, NPAD - N), (0, 0)))

    degp = _sc_degree(dst, w).reshape(NC, NPAD, 1)
    dinv, g = _tc_prolog(degp, xp)

    aggp = _sc_aggregate(g, src, dst, w, 128)
    g = _tc_layer1(aggp, g, dinv, W1, b1.reshape(1, -1), W2)

    for Wn, b in ((W3, b2), (W4, b3), (W5, b4), (W6, b5), (W7, b6)):
        aggp = _sc_aggregate(g, src, dst, w, g.shape[1])
        g = _tc_layer_mid(aggp, g, dinv, b.reshape(1, -1), Wn)

    aggp = _sc_aggregate(g, src, dst, w, 32)
    h = _tc_layer_last(aggp, g, dinv, b7.reshape(1, -1))
    return h[:N]


# speedup vs baseline: 10.6636x; 1.4262x over previous
import functools

import jax
import jax.numpy as jnp
from jax import lax
from jax.experimental import pallas as pl
from jax.experimental.pallas import tpu as pltpu
from jax.experimental.pallas import tpu_sc as plsc

N = 10000
NPAD = 10240
E = 320000
NC, NS, LANES = 2, 16, 16
NW = NC * NS
BLK = 128
NBUF = 4
NB = 80
NG = NB // NBUF
EPW = NB * BLK
E_PAD = EPW * NW
RPT = NPAD // NS
RB = 1280


def _sc_degree(dst_i, w_f):
    mesh = plsc.VectorSubcoreMesh(core_axis_name="c", subcore_axis_name="s")

    @functools.partial(
        pl.kernel,
        out_type=jax.ShapeDtypeStruct((NC, NPAD), jnp.float32),
        mesh=mesh,
        scratch_types=[
            pltpu.VMEM((NB, BLK), jnp.int32),
            pltpu.VMEM((NB, BLK), jnp.float32),
            pltpu.VMEM((RPT,), jnp.float32),
            pltpu.VMEM_SHARED((NPAD,), jnp.float32),
            pltpu.SemaphoreType.DMA,
        ],
    )
    def deg_kernel(dst_ref, w_ref, out_ref, di_all, w_all, z_v, acc_sh, sem):
        cid = lax.axis_index("c")
        sid = lax.axis_index("s")
        wid = cid * NS + sid

        pltpu.sync_copy(dst_ref.at[pl.ds(wid * NB, NB)], di_all)
        pltpu.sync_copy(w_ref.at[pl.ds(wid * NB, NB)], w_all)

        def zbody(i, _):
            z_v[pl.ds(i * LANES, LANES)] = jnp.zeros((LANES,), jnp.float32)
            return 0

        lax.fori_loop(0, RPT // LANES, zbody, 0)
        pltpu.sync_copy(z_v, acc_sh.at[pl.ds(sid * RPT, RPT)])
        plsc.subcore_barrier()

        def blk(j, _):
            pltpu.async_copy(w_all.at[j], acc_sh.at[di_all.at[j]], sem,
                             add=True)
            return 0

        lax.fori_loop(0, NB, blk, 0)

        def drain(j, _):
            pltpu.make_async_copy(
                w_all.at[j], acc_sh.at[di_all.at[j]], sem).wait()
            return 0

        lax.fori_loop(0, NB, drain, 0)
        plsc.subcore_barrier()
        pltpu.sync_copy(
            acc_sh.at[pl.ds(sid * RPT, RPT)],
            out_ref.at[cid, pl.ds(sid * RPT, RPT)],
        )

    return deg_kernel(dst_i, w_f)


def _sc_aggregate(g, src_i, dst_i, w_f, d):
    nbuf = 2 if d > 64 else NBUF
    ng = NB // nbuf
    mesh = plsc.VectorSubcoreMesh(core_axis_name="c", subcore_axis_name="s")

    @functools.partial(
        pl.kernel,
        out_type=jax.ShapeDtypeStruct((NC, NPAD, d), jnp.float32),
        mesh=mesh,
        scratch_types=[
            pltpu.VMEM((NB, BLK), jnp.int32),
            pltpu.VMEM((nbuf, BLK), jnp.int32),
            pltpu.VMEM((nbuf, BLK), jnp.float32),
            [pltpu.VMEM((BLK, d), jnp.float32) for _ in range(nbuf)],
            pltpu.VMEM_SHARED((NPAD, d), jnp.float32),
            [pltpu.SemaphoreType.DMA for _ in range(nbuf)],
            [pltpu.SemaphoreType.DMA for _ in range(nbuf)],
        ],
        compiler_params=pltpu.CompilerParams(use_tc_tiling_on_sc=False),
    )
    def agg_kernel(g_ref, src_ref, dst_ref, w_ref, out_ref,
                   si_all, di_ring, w_ring, rows, acc_sh, gsems, isems):
        cid = lax.axis_index("c")
        sid = lax.axis_index("s")
        wid = cid * NS + sid
        base = wid * NB

        pltpu.sync_copy(src_ref.at[pl.ds(base, NB)], si_all)

        def zbody(i, _):
            for c in range(d // LANES):
                rows[0][i, pl.ds(c * LANES, LANES)] = (
                    jnp.zeros((LANES,), jnp.float32))
            return 0

        lax.fori_loop(0, BLK, zbody, 0)
        for t in range(RPT // BLK):
            pltpu.sync_copy(rows[0], acc_sh.at[pl.ds(sid * RPT + t * BLK, BLK)])

        for b in range(nbuf):
            pltpu.async_copy(dst_ref.at[base + b], di_ring.at[b], isems[b])
            pltpu.async_copy(w_ref.at[base + b], w_ring.at[b], isems[b])
            pltpu.async_copy(g_ref.at[si_all.at[b]], rows[b], gsems[b])
        plsc.subcore_barrier()

        def group(jj, _):
            for b in range(nbuf):
                j = jj * nbuf + b
                pltpu.make_async_copy(
                    dst_ref.at[base + j], di_ring.at[b], isems[b]).wait()
                pltpu.make_async_copy(
                    w_ref.at[base + j], w_ring.at[b], isems[b]).wait()
                pltpu.make_async_copy(
                    g_ref.at[si_all.at[j]], rows[b], gsems[b]).wait()

                def ebody(gidx, _):
                    wv = w_ring[b, pl.ds(gidx * LANES, LANES)]
                    for l in range(LANES):
                        i = gidx * LANES + l
                        s = wv[l]
                        for c in range(d // LANES):
                            sl = pl.ds(c * LANES, LANES)
                            rows[b][i, sl] = rows[b][i, sl] * s
                    return 0

                lax.fori_loop(0, BLK // LANES, ebody, 0)
                pltpu.sync_copy(rows[b], acc_sh.at[di_ring.at[b]], add=True)

                @pl.when(jj < ng - 1)
                def _prefetch():
                    pltpu.async_copy(dst_ref.at[base + j + nbuf],
                                     di_ring.at[b], isems[b])
                    pltpu.async_copy(w_ref.at[base + j + nbuf],
                                     w_ring.at[b], isems[b])
                    pltpu.async_copy(
                        g_ref.at[si_all.at[j + nbuf]], rows[b], gsems[b])
            return 0

        lax.fori_loop(0, ng, group, 0)
        plsc.subcore_barrier()
        pltpu.sync_copy(
            acc_sh.at[pl.ds(sid * RPT, RPT)],
            out_ref.at[cid, pl.ds(sid * RPT, RPT)],
        )

    return agg_kernel(g, src_i, dst_i, w_f)


def _tc_prolog(degp, xp):
    def body(degp_ref, x_ref, dinv_ref, g_ref):
        deg = degp_ref[0] + degp_ref[1] + 1.0
        dinv = jnp.where(deg > 0, lax.rsqrt(deg), 0.0)
        dinv_ref[...] = dinv
        g_ref[...] = x_ref[...] * dinv

    return pl.pallas_call(
        body,
        grid=(NPAD // RB,),
        in_specs=[
            pl.BlockSpec((2, RB, 1), lambda i: (0, i, 0)),
            pl.BlockSpec((RB, 128), lambda i: (i, 0)),
        ],
        out_specs=[
            pl.BlockSpec((RB, 1), lambda i: (i, 0)),
            pl.BlockSpec((RB, 128), lambda i: (i, 0)),
        ],
        out_shape=[
            jax.ShapeDtypeStruct((NPAD, 1), jnp.float32),
            jax.ShapeDtypeStruct((NPAD, 128), jnp.float32),
        ],
    )(degp, xp)


def _tc_layer1(aggp, g, dinv, W1, b1, W2):
    def body(a_ref, g_ref, dinv_ref, w1_ref, b1_ref, w2_ref, out_ref):
        dinv = dinv_ref[...]
        s = dinv * (a_ref[0] + a_ref[1] + g_ref[...])
        h = jnp.maximum(
            jnp.dot(s, w1_ref[...], preferred_element_type=jnp.float32)
            + b1_ref[...], 0.0)
        out_ref[...] = dinv * jnp.dot(
            h, w2_ref[...], preferred_element_type=jnp.float32)

    return pl.pallas_call(
        body,
        grid=(NPAD // RB,),
        in_specs=[
            pl.BlockSpec((2, RB, 128), lambda i: (0, i, 0)),
            pl.BlockSpec((RB, 128), lambda i: (i, 0)),
            pl.BlockSpec((RB, 1), lambda i: (i, 0)),
            pl.BlockSpec((128, 256), lambda i: (0, 0)),
            pl.BlockSpec((1, 256), lambda i: (0, 0)),
            pl.BlockSpec((256, 128), lambda i: (0, 0)),
        ],
        out_specs=pl.BlockSpec((RB, 128), lambda i: (i, 0)),
        out_shape=jax.ShapeDtypeStruct((NPAD, 128), jnp.float32),
    )(aggp, g, dinv, W1, b1, W2)


def _tc_layer_mid(aggp, g, dinv, b, Wn):
    d = g.shape[1]
    dn = Wn.shape[1]

    def body(a_ref, g_ref, dinv_ref, b_ref, wn_ref, out_ref):
        dinv = dinv_ref[...]
        h = jnp.maximum(
            dinv * (a_ref[0] + a_ref[1] + g_ref[...]) + b_ref[...], 0.0)
        out_ref[...] = dinv * jnp.dot(
            h, wn_ref[...], preferred_element_type=jnp.float32)

    return pl.pallas_call(
        body,
        grid=(NPAD // RB,),
        in_specs=[
            pl.BlockSpec((2, RB, d), lambda i: (0, i, 0)),
            pl.BlockSpec((RB, d), lambda i: (i, 0)),
            pl.BlockSpec((RB, 1), lambda i: (i, 0)),
            pl.BlockSpec((1, d), lambda i: (0, 0)),
            pl.BlockSpec((d, dn), lambda i: (0, 0)),
        ],
        out_specs=pl.BlockSpec((RB, dn), lambda i: (i, 0)),
        out_shape=jax.ShapeDtypeStruct((NPAD, dn), jnp.float32),
    )(aggp, g, dinv, b, Wn)


def _tc_layer_last(aggp, g, dinv, b):
    d = g.shape[1]

    def body(a_ref, g_ref, dinv_ref, b_ref, out_ref):
        out_ref[...] = jnp.maximum(
            dinv_ref[...] * (a_ref[0] + a_ref[1] + g_ref[...]) + b_ref[...],
            0.0)

    return pl.pallas_call(
        body,
        grid=(NPAD // RB,),
        in_specs=[
            pl.BlockSpec((2, RB, d), lambda i: (0, i, 0)),
            pl.BlockSpec((RB, d), lambda i: (i, 0)),
            pl.BlockSpec((RB, 1), lambda i: (i, 0)),
            pl.BlockSpec((1, d), lambda i: (0, 0)),
        ],
        out_specs=pl.BlockSpec((RB, d), lambda i: (i, 0)),
        out_shape=jax.ShapeDtypeStruct((NPAD, d), jnp.float32),
    )(aggp, g, dinv, b)


def kernel(x, edge_index, edge_weight, W1, b1, W2, b2, W3, b3, W4, b4,
           W5, b5, W6, b6, W7, b7):
    src = edge_index[0].astype(jnp.int32)
    dst = edge_index[1].astype(jnp.int32)
    w = edge_weight.astype(jnp.float32)
    pad = E_PAD - E
    src = jnp.concatenate([src, jnp.zeros((pad,), jnp.int32)])
    dst = jnp.concatenate([dst, jnp.zeros((pad,), jnp.int32)])
    w = jnp.concatenate([w, jnp.zeros((pad,), jnp.float32)])
    src = src.reshape(NW * NB, BLK)
    dst = dst.reshape(NW * NB, BLK)
    w = w.reshape(NW * NB, BLK)
    xp = jnp.pad(x, ((0, NPAD - N), (0, 0)))

    degp = _sc_degree(dst, w).reshape(NC, NPAD, 1)
    dinv, g = _tc_prolog(degp, xp)

    aggp = _sc_aggregate(g, src, dst, w, 128)
    g = _tc_layer1(aggp, g, dinv, W1, b1.reshape(1, -1), W2)

    for Wn, b in ((W3, b2), (W4, b3), (W5, b4), (W6, b5), (W7, b6)):
        aggp = _sc_aggregate(g, src, dst, w, g.shape[1])
        g = _tc_layer_mid(aggp, g, dinv, b.reshape(1, -1), Wn)

    aggp = _sc_aggregate(g, src, dst, w, 32)
    h = _tc_layer_last(aggp, g, dinv, b7.reshape(1, -1))
    return h[:N]


# speedup vs baseline: 15.6983x; 1.4721x over previous
import functools

import jax
import jax.numpy as jnp
from jax import lax
from jax.experimental import pallas as pl
from jax.experimental.pallas import tpu as pltpu
from jax.experimental.pallas import tpu_sc as plsc

N = 10000
NPAD = 10240
E = 320000
NC, NS, LANES = 2, 16, 16
NW = NC * NS
BLK = 128
NBUF = 4
NB0, NB1 = 120, 40
TBLK = NS * (NB0 + NB1)
E_PAD = TBLK * BLK
RPT = NPAD // NS
RB = 1280


def _sc_degree(dst_i, w_f):
    mesh = plsc.VectorSubcoreMesh(core_axis_name="c", subcore_axis_name="s")

    @functools.partial(
        pl.kernel,
        out_type=jax.ShapeDtypeStruct((NC, NPAD), jnp.float32),
        mesh=mesh,
        scratch_types=[
            pltpu.VMEM((NB0, BLK), jnp.int32),
            pltpu.VMEM((NB0, BLK), jnp.float32),
            pltpu.VMEM((RPT,), jnp.float32),
            pltpu.VMEM_SHARED((NPAD,), jnp.float32),
            pltpu.SemaphoreType.DMA,
        ],
    )
    def deg_kernel(dst_ref, w_ref, out_ref, di_all, w_all, z_v, acc_sh, sem):
        cid = lax.axis_index("c")
        sid = lax.axis_index("s")
        nbc = jnp.where(cid == 0, NB0, NB1)
        base = cid * (NS * NB0) + sid * nbc
        off = NB0 - nbc

        pltpu.sync_copy(dst_ref.at[pl.ds(base - off, NB0)], di_all)
        pltpu.sync_copy(w_ref.at[pl.ds(base - off, NB0)], w_all)

        def zbody(i, _):
            z_v[pl.ds(i * LANES, LANES)] = jnp.zeros((LANES,), jnp.float32)
            return 0

        lax.fori_loop(0, RPT // LANES, zbody, 0)
        pltpu.sync_copy(z_v, acc_sh.at[pl.ds(sid * RPT, RPT)])
        plsc.subcore_barrier()

        def blk(j, _):
            pltpu.async_copy(w_all.at[off + j], acc_sh.at[di_all.at[off + j]],
                             sem, add=True)
            return 0

        lax.fori_loop(0, nbc, blk, 0)

        def drain(j, _):
            pltpu.make_async_copy(
                w_all.at[off + j], acc_sh.at[di_all.at[off + j]], sem).wait()
            return 0

        lax.fori_loop(0, nbc, drain, 0)
        plsc.subcore_barrier()
        pltpu.sync_copy(
            acc_sh.at[pl.ds(sid * RPT, RPT)],
            out_ref.at[cid, pl.ds(sid * RPT, RPT)],
        )

    return deg_kernel(dst_i, w_f)


def _sc_aggregate(g, src_i, dst_i, w_f, d):
    nbuf = 2 if d > 64 else NBUF
    mesh = plsc.VectorSubcoreMesh(core_axis_name="c", subcore_axis_name="s")

    @functools.partial(
        pl.kernel,
        out_type=jax.ShapeDtypeStruct((NC, NPAD, d), jnp.float32),
        mesh=mesh,
        scratch_types=[
            pltpu.VMEM((NB0, BLK), jnp.int32),
            pltpu.VMEM((nbuf, BLK), jnp.int32),
            pltpu.VMEM((nbuf, BLK), jnp.float32),
            [pltpu.VMEM((BLK, d), jnp.float32) for _ in range(nbuf)],
            pltpu.VMEM_SHARED((NPAD, d), jnp.float32),
            [pltpu.SemaphoreType.DMA for _ in range(nbuf)],
            [pltpu.SemaphoreType.DMA for _ in range(nbuf)],
        ],
        compiler_params=pltpu.CompilerParams(use_tc_tiling_on_sc=False),
    )
    def agg_kernel(g_ref, src_ref, dst_ref, w_ref, out_ref,
                   si_all, di_ring, w_ring, rows, acc_sh, gsems, isems):
        cid = lax.axis_index("c")
        sid = lax.axis_index("s")
        nbc = jnp.where(cid == 0, NB0, NB1)
        base = cid * (NS * NB0) + sid * nbc
        ng = nbc // nbuf
        off = NB0 - nbc

        pltpu.sync_copy(src_ref.at[pl.ds(base - off, NB0)], si_all)

        def zbody(i, _):
            for c in range(d // LANES):
                rows[0][i, pl.ds(c * LANES, LANES)] = (
                    jnp.zeros((LANES,), jnp.float32))
            return 0

        lax.fori_loop(0, BLK, zbody, 0)
        for t in range(RPT // BLK):
            pltpu.sync_copy(rows[0], acc_sh.at[pl.ds(sid * RPT + t * BLK, BLK)])

        for b in range(nbuf):
            pltpu.async_copy(dst_ref.at[base + b], di_ring.at[b], isems[b])
            pltpu.async_copy(w_ref.at[base + b], w_ring.at[b], isems[b])
            pltpu.async_copy(g_ref.at[si_all.at[b]], rows[b], gsems[b])
        plsc.subcore_barrier()

        def group(jj, _):
            for b in range(nbuf):
                j = jj * nbuf + b
                pltpu.make_async_copy(
                    dst_ref.at[base + j], di_ring.at[b], isems[b]).wait()
                pltpu.make_async_copy(
                    w_ref.at[base + j], w_ring.at[b], isems[b]).wait()
                pltpu.make_async_copy(
                    g_ref.at[si_all.at[j]], rows[b], gsems[b]).wait()

                def ebody(gidx, _):
                    wv = w_ring[b, pl.ds(gidx * LANES, LANES)]
                    for l in range(LANES):
                        i = gidx * LANES + l
                        s = wv[l]
                        for c in range(d // LANES):
                            sl = pl.ds(c * LANES, LANES)
                            rows[b][i, sl] = rows[b][i, sl] * s
                    return 0

                lax.fori_loop(0, BLK // LANES, ebody, 0)
                pltpu.sync_copy(rows[b], acc_sh.at[di_ring.at[b]], add=True)

                @pl.when(jj < ng - 1)
                def _prefetch():
                    pltpu.async_copy(dst_ref.at[base + j + nbuf],
                                     di_ring.at[b], isems[b])
                    pltpu.async_copy(w_ref.at[base + j + nbuf],
                                     w_ring.at[b], isems[b])
                    pltpu.async_copy(
                        g_ref.at[si_all.at[j + nbuf]], rows[b], gsems[b])
            return 0

        lax.fori_loop(0, ng, group, 0)
        plsc.subcore_barrier()
        pltpu.sync_copy(
            acc_sh.at[pl.ds(sid * RPT, RPT)],
            out_ref.at[cid, pl.ds(sid * RPT, RPT)],
        )

    return agg_kernel(g, src_i, dst_i, w_f)


def _tc_prolog(degp, xp):
    def body(degp_ref, x_ref, dinv_ref, g_ref):
        deg = degp_ref[0] + degp_ref[1] + 1.0
        dinv = jnp.where(deg > 0, lax.rsqrt(deg), 0.0)
        dinv_ref[...] = dinv
        g_ref[...] = x_ref[...] * dinv

    return pl.pallas_call(
        body,
        grid=(NPAD // RB,),
        in_specs=[
            pl.BlockSpec((2, RB, 1), lambda i: (0, i, 0)),
            pl.BlockSpec((RB, 128), lambda i: (i, 0)),
        ],
        out_specs=[
            pl.BlockSpec((RB, 1), lambda i: (i, 0)),
            pl.BlockSpec((RB, 128), lambda i: (i, 0)),
        ],
        out_shape=[
            jax.ShapeDtypeStruct((NPAD, 1), jnp.float32),
            jax.ShapeDtypeStruct((NPAD, 128), jnp.float32),
        ],
    )(degp, xp)


def _tc_layer1(aggp, g, dinv, W1, b1, W2):
    def body(a_ref, g_ref, dinv_ref, w1_ref, b1_ref, w2_ref, out_ref):
        dinv = dinv_ref[...]
        s = dinv * (a_ref[0] + a_ref[1] + g_ref[...])
        h = jnp.maximum(
            jnp.dot(s, w1_ref[...], preferred_element_type=jnp.float32)
            + b1_ref[...], 0.0)
        out_ref[...] = dinv * jnp.dot(
            h, w2_ref[...], preferred_element_type=jnp.float32)

    return pl.pallas_call(
        body,
        grid=(NPAD // RB,),
        in_specs=[
            pl.BlockSpec((2, RB, 128), lambda i: (0, i, 0)),
            pl.BlockSpec((RB, 128), lambda i: (i, 0)),
            pl.BlockSpec((RB, 1), lambda i: (i, 0)),
            pl.BlockSpec((128, 256), lambda i: (0, 0)),
            pl.BlockSpec((1, 256), lambda i: (0, 0)),
            pl.BlockSpec((256, 128), lambda i: (0, 0)),
        ],
        out_specs=pl.BlockSpec((RB, 128), lambda i: (i, 0)),
        out_shape=jax.ShapeDtypeStruct((NPAD, 128), jnp.float32),
    )(aggp, g, dinv, W1, b1, W2)


def _tc_layer_mid(aggp, g, dinv, b, Wn):
    d = g.shape[1]
    dn = Wn.shape[1]

    def body(a_ref, g_ref, dinv_ref, b_ref, wn_ref, out_ref):
        dinv = dinv_ref[...]
        h = jnp.maximum(
            dinv * (a_ref[0] + a_ref[1] + g_ref[...]) + b_ref[...], 0.0)
        out_ref[...] = dinv * jnp.dot(
            h, wn_ref[...], preferred_element_type=jnp.float32)

    return pl.pallas_call(
        body,
        grid=(NPAD // RB,),
        in_specs=[
            pl.BlockSpec((2, RB, d), lambda i: (0, i, 0)),
            pl.BlockSpec((RB, d), lambda i: (i, 0)),
            pl.BlockSpec((RB, 1), lambda i: (i, 0)),
            pl.BlockSpec((1, d), lambda i: (0, 0)),
            pl.BlockSpec((d, dn), lambda i: (0, 0)),
        ],
        out_specs=pl.BlockSpec((RB, dn), lambda i: (i, 0)),
        out_shape=jax.ShapeDtypeStruct((NPAD, dn), jnp.float32),
    )(aggp, g, dinv, b, Wn)


def _tc_layer_last(aggp, g, dinv, b):
    d = g.shape[1]

    def body(a_ref, g_ref, dinv_ref, b_ref, out_ref):
        out_ref[...] = jnp.maximum(
            dinv_ref[...] * (a_ref[0] + a_ref[1] + g_ref[...]) + b_ref[...],
            0.0)

    return pl.pallas_call(
        body,
        grid=(NPAD // RB,),
        in_specs=[
            pl.BlockSpec((2, RB, d), lambda i: (0, i, 0)),
            pl.BlockSpec((RB, d), lambda i: (i, 0)),
            pl.BlockSpec((RB, 1), lambda i: (i, 0)),
            pl.BlockSpec((1, d), lambda i: (0, 0)),
        ],
        out_specs=pl.BlockSpec((RB, d), lambda i: (i, 0)),
        out_shape=jax.ShapeDtypeStruct((NPAD, d), jnp.float32),
    )(aggp, g, dinv, b)


def kernel(x, edge_index, edge_weight, W1, b1, W2, b2, W3, b3, W4, b4,
           W5, b5, W6, b6, W7, b7):
    src = edge_index[0].astype(jnp.int32)
    dst = edge_index[1].astype(jnp.int32)
    w = edge_weight.astype(jnp.float32)
    pad = E_PAD - E
    src = jnp.concatenate([src, jnp.zeros((pad,), jnp.int32)])
    dst = jnp.concatenate([dst, jnp.zeros((pad,), jnp.int32)])
    w = jnp.concatenate([w, jnp.zeros((pad,), jnp.float32)])
    src = src.reshape(TBLK, BLK)
    dst = dst.reshape(TBLK, BLK)
    w = w.reshape(TBLK, BLK)
    xp = jnp.pad(x, ((0, NPAD - N), (0, 0)))

    degp = _sc_degree(dst, w).reshape(NC, NPAD, 1)
    dinv, g = _tc_prolog(degp, xp)

    aggp = _sc_aggregate(g, src, dst, w, 128)
    g = _tc_layer1(aggp, g, dinv, W1, b1.reshape(1, -1), W2)

    for Wn, b in ((W3, b2), (W4, b3), (W5, b4), (W6, b5), (W7, b6)):
        aggp = _sc_aggregate(g, src, dst, w, g.shape[1])
        g = _tc_layer_mid(aggp, g, dinv, b.reshape(1, -1), Wn)

    aggp = _sc_aggregate(g, src, dst, w, 32)
    h = _tc_layer_last(aggp, g, dinv, b7.reshape(1, -1))
    return h[:N]
